# Initial kernel scaffold; baseline (speedup 1.0000x reference)
import jax
import jax.numpy as jnp
from jax.experimental import pallas as pl


def kernel(x_image, x_text, edge_index_image, edge_index_text, W_img, a_l_img, a_r_img, b_img, W_txt, a_l_txt, a_r_txt, b_txt):
    raise NotImplementedError("write your pallas kernel here")


# speedup vs baseline: 18.5124x; 18.5124x over previous
import functools

import jax
import jax.numpy as jnp
from jax import lax
from jax.experimental import pallas as pl
from jax.experimental.pallas import tpu as pltpu
from jax.experimental.pallas import tpu_sc as plsc

N = 10000
D = 128
E = 160000
B = 128
NT = 16
VROWS = E // B
RPT = 80
EROWS = RPT * NT
NP = 10240
OPT = NP // NT
ECH = 64


def _prep_body(xi, xt, wi, ali, ari, bi, wt, alt, art, bt,
               h1, h2, el1, er1, el2, er2, f1, f2):
    h1[...] = jnp.dot(xi[...], wi[...], preferred_element_type=jnp.float32)
    el1[...] = jnp.dot(h1[...], ali[...], preferred_element_type=jnp.float32)
    er1[...] = jnp.dot(xt[...], jnp.dot(wi[...], ari[...]),
                       preferred_element_type=jnp.float32)
    f1[...] = xt[...] + bi[...]
    h2[...] = jnp.dot(xt[...], wt[...], preferred_element_type=jnp.float32)
    el2[...] = jnp.dot(h2[...], alt[...], preferred_element_type=jnp.float32)
    er2[...] = jnp.dot(xi[...], jnp.dot(wt[...], art[...]),
                       preferred_element_type=jnp.float32)
    f2[...] = xi[...] + bt[...]


_prep = pl.pallas_call(
    _prep_body,
    out_shape=[
        jax.ShapeDtypeStruct((N, D), jnp.float32),
        jax.ShapeDtypeStruct((N, D), jnp.float32),
        jax.ShapeDtypeStruct((N, 1), jnp.float32),
        jax.ShapeDtypeStruct((N, 1), jnp.float32),
        jax.ShapeDtypeStruct((N, 1), jnp.float32),
        jax.ShapeDtypeStruct((N, 1), jnp.float32),
        jax.ShapeDtypeStruct((N, D), jnp.float32),
        jax.ShapeDtypeStruct((N, D), jnp.float32),
    ],
)

_mesh = plsc.VectorSubcoreMesh(core_axis_name="c", subcore_axis_name="s")


@functools.partial(
    pl.kernel,
    out_type=[
        jax.ShapeDtypeStruct((NP, D), jnp.float32),
        jax.ShapeDtypeStruct((NP, D), jnp.float32),
    ],
    mesh=_mesh,
    compiler_params=pltpu.CompilerParams(use_tc_tiling_on_sc=False,
                                         needs_layout_passes=False),
    scratch_types=[
        pltpu.VMEM((N,), jnp.float32),
        pltpu.VMEM((N,), jnp.float32),
        pltpu.VMEM((B,), jnp.int32),
        pltpu.VMEM((B,), jnp.int32),
        pltpu.VMEM((B, D), jnp.float32),
        pltpu.VMEM((B,), jnp.float32),
        pltpu.VMEM((B, 16), jnp.float32),
        pltpu.VMEM_SHARED((NP, D), jnp.float32),
        pltpu.VMEM_SHARED((NP, 16), jnp.float32),
    ],
)
def _edge(h1, l1, e1, s1, d1, f1, h2, l2, e2, s2, d2, f2, o1, o2,
          el_v, er_v, sidx_v, didx_v, rows_v, ex_v, exw_v, u_sh, s_sh):
    c = lax.axis_index("c")
    s = lax.axis_index("s")
    z16 = jnp.zeros((16,), jnp.float32)

    def do_conv(a_h, el_h, er_h, src_h, dst_h, feat_h, out_h):
        pltpu.sync_copy(el_h, el_v)
        pltpu.sync_copy(er_h, er_v)

        def zrow(r, carry):
            for j in range(D // 16):
                rows_v[r, pl.ds(j * 16, 16)] = z16
            exw_v[r, pl.ds(0, 16)] = z16
            return carry
        lax.fori_loop(0, B, zrow, 0)
        base = s * OPT
        for k in range(OPT // B):
            pltpu.sync_copy(rows_v, u_sh.at[pl.ds(base + k * B, B)])
            pltpu.sync_copy(exw_v, s_sh.at[pl.ds(base + k * B, B)])
        plsc.subcore_barrier()

        def batch(b, carry):
            row = s * RPT + b

            @pl.when(row < VROWS)
            def _():
                pltpu.sync_copy(src_h.at[row], sidx_v)
                pltpu.sync_copy(dst_h.at[row], didx_v)
                pltpu.sync_copy(a_h.at[sidx_v], rows_v)
                for j in range(B // 16):
                    sl = pl.ds(j * 16, 16)
                    s16 = sidx_v[sl]
                    d16 = didx_v[sl]
                    el16 = plsc.load_gather(el_v, [s16])
                    er16 = plsc.load_gather(er_v, [d16])
                    e = el16 + er16
                    e = jnp.where(e >= 0.0, e, 0.2 * e)
                    ex_v[sl] = jnp.exp(e)

                def scale(r, carry2):
                    exv = plsc.load_gather(ex_v, [jnp.full((16,), r, jnp.int32)])
                    for j in range(D // 16):
                        sl = pl.ds(j * 16, 16)
                        rows_v[r, sl] = rows_v[r, sl] * exv
                    exw_v[r, pl.ds(0, 16)] = exv
                    return carry2
                lax.fori_loop(0, B, scale, 0)
                pltpu.sync_copy(rows_v, u_sh.at[didx_v], add=True)
                pltpu.sync_copy(exw_v, s_sh.at[didx_v], add=True)
            return carry
        lax.fori_loop(0, RPT, batch, 0)
        plsc.subcore_barrier()

        def echunk(k, carry):
            rb = base + k * ECH
            pltpu.sync_copy(u_sh.at[pl.ds(rb, ECH)], rows_v.at[pl.ds(0, ECH)])
            pltpu.sync_copy(s_sh.at[pl.ds(rb, ECH)], exw_v.at[pl.ds(0, ECH)])
            pltpu.sync_copy(feat_h.at[pl.ds(rb, ECH)],
                            rows_v.at[pl.ds(ECH, ECH)])

            def erow(r, carry2):
                iv = 1.0 / (exw_v[r, pl.ds(0, 16)] + 1e-9)
                for j in range(D // 16):
                    sl = pl.ds(j * 16, 16)
                    v = rows_v[r, sl] * iv + rows_v[r + ECH, sl]
                    rows_v[r, sl] = jnp.where(v > 0.0, v, jnp.exp(v) - 1.0)
                return carry2
            lax.fori_loop(0, ECH, erow, 0)
            pltpu.sync_copy(rows_v.at[pl.ds(0, ECH)], out_h.at[pl.ds(rb, ECH)])
            return carry
        lax.fori_loop(0, OPT // ECH, echunk, 0)

    @pl.when(c == 0)
    def _():
        do_conv(h1, l1, e1, s1, d1, f1, o1)

    @pl.when(c == 1)
    def _():
        do_conv(h2, l2, e2, s2, d2, f2, o2)


def _pad_edges(idx):
    idx = idx.astype(jnp.int32)
    padlen = EROWS * B - E
    pad = jnp.arange(padlen, dtype=jnp.int32) % N
    return jnp.concatenate([idx, pad]).reshape(EROWS, B)


def kernel(x_image, x_text, edge_index_image, edge_index_text,
           W_img, a_l_img, a_r_img, b_img, W_txt, a_l_txt, a_r_txt, b_txt):
    h1, h2, el1, er1, el2, er2, f1, f2 = _prep(
        x_image, x_text,
        W_img, a_l_img.reshape(D, 1), a_r_img.reshape(D, 1),
        b_img.reshape(1, D),
        W_txt, a_l_txt.reshape(D, 1), a_r_txt.reshape(D, 1),
        b_txt.reshape(1, D),
    )
    s1 = _pad_edges(edge_index_image[0])
    d1 = _pad_edges(edge_index_image[1])
    s2 = _pad_edges(edge_index_text[0])
    d2 = _pad_edges(edge_index_text[1])
    fpad = jnp.zeros((NP - N, D), jnp.float32)
    ftp = jnp.concatenate([f1, fpad])
    fip = jnp.concatenate([f2, fpad])
    new_text, new_image = _edge(
        h1, el1.reshape(N), er1.reshape(N), s1, d1, ftp,
        h2, el2.reshape(N), er2.reshape(N), s2, d2, fip,
    )
    return (new_image[:N], new_text[:N])


# speedup vs baseline: 21.7887x; 1.1770x over previous
import functools

import jax
import jax.numpy as jnp
from jax import lax
from jax.experimental import pallas as pl
from jax.experimental.pallas import tpu as pltpu
from jax.experimental.pallas import tpu_sc as plsc

N = 10000
D = 128
E = 160000
B = 128
NT = 16
VROWS = E // B
RPT = 80
EROWS = RPT * NT
NP = 10240
OPT = NP // NT
ECH = 64


def _prep_body(xi, xt, wi, ali, ari, bi, wt, alt, art, bt,
               h1, h2, el1, er1, el2, er2, f1, f2):
    h1[...] = jnp.dot(xi[...], wi[...], preferred_element_type=jnp.float32)
    el1[...] = jnp.dot(h1[...], ali[...], preferred_element_type=jnp.float32)
    er1[...] = jnp.dot(xt[...], jnp.dot(wi[...], ari[...]),
                       preferred_element_type=jnp.float32)
    f1[...] = xt[...] + bi[...]
    h2[...] = jnp.dot(xt[...], wt[...], preferred_element_type=jnp.float32)
    el2[...] = jnp.dot(h2[...], alt[...], preferred_element_type=jnp.float32)
    er2[...] = jnp.dot(xi[...], jnp.dot(wt[...], art[...]),
                       preferred_element_type=jnp.float32)
    f2[...] = xi[...] + bt[...]


_prep = pl.pallas_call(
    _prep_body,
    out_shape=[
        jax.ShapeDtypeStruct((N, D), jnp.float32),
        jax.ShapeDtypeStruct((N, D), jnp.float32),
        jax.ShapeDtypeStruct((N, 1), jnp.float32),
        jax.ShapeDtypeStruct((N, 1), jnp.float32),
        jax.ShapeDtypeStruct((N, 1), jnp.float32),
        jax.ShapeDtypeStruct((N, 1), jnp.float32),
        jax.ShapeDtypeStruct((N, D), jnp.float32),
        jax.ShapeDtypeStruct((N, D), jnp.float32),
    ],
)

_mesh = plsc.VectorSubcoreMesh(core_axis_name="c", subcore_axis_name="s")


@functools.partial(
    pl.kernel,
    out_type=[
        jax.ShapeDtypeStruct((NP, D), jnp.float32),
        jax.ShapeDtypeStruct((NP, D), jnp.float32),
    ],
    mesh=_mesh,
    compiler_params=pltpu.CompilerParams(use_tc_tiling_on_sc=False,
                                         needs_layout_passes=False),
    scratch_types=[
        pltpu.VMEM((N,), jnp.float32),
        pltpu.VMEM((N,), jnp.float32),
        pltpu.VMEM((B,), jnp.int32),
        pltpu.VMEM((B,), jnp.int32),
        pltpu.VMEM((B, D), jnp.float32),
        pltpu.VMEM((B,), jnp.float32),
        pltpu.VMEM((B, 16), jnp.float32),
        pltpu.VMEM_SHARED((NP, D), jnp.float32),
        pltpu.VMEM_SHARED((NP, 16), jnp.float32),
    ],
)
def _edge(h1, l1, e1, s1, d1, f1, h2, l2, e2, s2, d2, f2, o1, o2,
          el_v, er_v, sidx_v, didx_v, rows_v, ex_v, exw_v, u_sh, s_sh):
    c = lax.axis_index("c")
    s = lax.axis_index("s")
    z16 = jnp.zeros((16,), jnp.float32)

    def do_conv(a_h, el_h, er_h, src_h, dst_h, feat_h, out_h):
        pltpu.sync_copy(el_h, el_v)
        pltpu.sync_copy(er_h, er_v)

        @plsc.parallel_loop(0, B, unroll=4)
        def _(r):
            for j in range(D // 16):
                rows_v[r, pl.ds(j * 16, 16)] = z16
            exw_v[r, pl.ds(0, 16)] = z16
        base = s * OPT
        for k in range(OPT // B):
            pltpu.sync_copy(rows_v, u_sh.at[pl.ds(base + k * B, B)])
            pltpu.sync_copy(exw_v, s_sh.at[pl.ds(base + k * B, B)])
        plsc.subcore_barrier()

        def batch(b, carry):
            row = s * RPT + b

            @pl.when(row < VROWS)
            def _():
                pltpu.sync_copy(src_h.at[row], sidx_v)
                pltpu.sync_copy(dst_h.at[row], didx_v)
                pltpu.sync_copy(a_h.at[sidx_v], rows_v)
                for j in range(B // 16):
                    sl = pl.ds(j * 16, 16)
                    s16 = sidx_v[sl]
                    d16 = didx_v[sl]
                    el16 = plsc.load_gather(el_v, [s16])
                    er16 = plsc.load_gather(er_v, [d16])
                    e = el16 + er16
                    e = jnp.where(e >= 0.0, e, 0.2 * e)
                    ex_v[sl] = jnp.exp(e)

                @plsc.parallel_loop(0, B)
                def _(r):
                    exv = plsc.load_gather(ex_v, [jnp.full((16,), r, jnp.int32)])
                    for j in range(D // 16):
                        sl = pl.ds(j * 16, 16)
                        rows_v[r, sl] = rows_v[r, sl] * exv
                    exw_v[r, pl.ds(0, 16)] = exv
                pltpu.sync_copy(rows_v, u_sh.at[didx_v], add=True)
                pltpu.sync_copy(exw_v, s_sh.at[didx_v], add=True)
            return carry
        lax.fori_loop(0, RPT, batch, 0)
        plsc.subcore_barrier()

        def echunk(k, carry):
            rb = base + k * ECH
            pltpu.sync_copy(u_sh.at[pl.ds(rb, ECH)], rows_v.at[pl.ds(0, ECH)])
            pltpu.sync_copy(s_sh.at[pl.ds(rb, ECH)], exw_v.at[pl.ds(0, ECH)])
            pltpu.sync_copy(feat_h.at[pl.ds(rb, ECH)],
                            rows_v.at[pl.ds(ECH, ECH)])

            @plsc.parallel_loop(0, ECH)
            def _(r):
                iv = 1.0 / (exw_v[r, pl.ds(0, 16)] + 1e-9)
                for j in range(D // 16):
                    sl = pl.ds(j * 16, 16)
                    v = rows_v[r, sl] * iv + rows_v[r + ECH, sl]
                    rows_v[r, sl] = jnp.where(v > 0.0, v, jnp.exp(v) - 1.0)
            pltpu.sync_copy(rows_v.at[pl.ds(0, ECH)], out_h.at[pl.ds(rb, ECH)])
            return carry
        lax.fori_loop(0, OPT // ECH, echunk, 0)

    @pl.when(c == 0)
    def _():
        do_conv(h1, l1, e1, s1, d1, f1, o1)

    @pl.when(c == 1)
    def _():
        do_conv(h2, l2, e2, s2, d2, f2, o2)


def _pad_edges(idx):
    idx = idx.astype(jnp.int32)
    padlen = EROWS * B - E
    pad = jnp.arange(padlen, dtype=jnp.int32) % N
    return jnp.concatenate([idx, pad]).reshape(EROWS, B)


def kernel(x_image, x_text, edge_index_image, edge_index_text,
           W_img, a_l_img, a_r_img, b_img, W_txt, a_l_txt, a_r_txt, b_txt):
    h1, h2, el1, er1, el2, er2, f1, f2 = _prep(
        x_image, x_text,
        W_img, a_l_img.reshape(D, 1), a_r_img.reshape(D, 1),
        b_img.reshape(1, D),
        W_txt, a_l_txt.reshape(D, 1), a_r_txt.reshape(D, 1),
        b_txt.reshape(1, D),
    )
    s1 = _pad_edges(edge_index_image[0])
    d1 = _pad_edges(edge_index_image[1])
    s2 = _pad_edges(edge_index_text[0])
    d2 = _pad_edges(edge_index_text[1])
    fpad = jnp.zeros((NP - N, D), jnp.float32)
    ftp = jnp.concatenate([f1, fpad])
    fip = jnp.concatenate([f2, fpad])
    new_text, new_image = _edge(
        h1, el1.reshape(N), er1.reshape(N), s1, d1, ftp,
        h2, el2.reshape(N), er2.reshape(N), s2, d2, fip,
    )
    return (new_image[:N], new_text[:N])


# speedup vs baseline: 23.8473x; 1.0945x over previous
import functools

import jax
import jax.numpy as jnp
from jax import lax
from jax.experimental import pallas as pl
from jax.experimental.pallas import tpu as pltpu
from jax.experimental.pallas import tpu_sc as plsc

N = 10000
D = 128
E = 160000
B = 64
NT = 16
VROWS = E // B
RPT = 157
EROWS = RPT * NT
NPAIR = (RPT + 1) // 2
NP = 10240
OPT = NP // NT
ECH = 64


def _prep_body(xi, xt, wi, ali, ari, bi, wt, alt, art, bt,
               h1, h2, el1, er1, el2, er2, f1, f2):
    h1[...] = jnp.dot(xi[...], wi[...], preferred_element_type=jnp.float32)
    el1[...] = jnp.dot(h1[...], ali[...], preferred_element_type=jnp.float32)
    er1[...] = jnp.dot(xt[...], jnp.dot(wi[...], ari[...]),
                       preferred_element_type=jnp.float32)
    f1[...] = xt[...] + bi[...]
    h2[...] = jnp.dot(xt[...], wt[...], preferred_element_type=jnp.float32)
    el2[...] = jnp.dot(h2[...], alt[...], preferred_element_type=jnp.float32)
    er2[...] = jnp.dot(xi[...], jnp.dot(wt[...], art[...]),
                       preferred_element_type=jnp.float32)
    f2[...] = xi[...] + bt[...]


_prep = pl.pallas_call(
    _prep_body,
    out_shape=[
        jax.ShapeDtypeStruct((N, D), jnp.float32),
        jax.ShapeDtypeStruct((N, D), jnp.float32),
        jax.ShapeDtypeStruct((N, 1), jnp.float32),
        jax.ShapeDtypeStruct((N, 1), jnp.float32),
        jax.ShapeDtypeStruct((N, 1), jnp.float32),
        jax.ShapeDtypeStruct((N, 1), jnp.float32),
        jax.ShapeDtypeStruct((N, D), jnp.float32),
        jax.ShapeDtypeStruct((N, D), jnp.float32),
    ],
)

_mesh = plsc.VectorSubcoreMesh(core_axis_name="c", subcore_axis_name="s")


@functools.partial(
    pl.kernel,
    out_type=[
        jax.ShapeDtypeStruct((NP, D), jnp.float32),
        jax.ShapeDtypeStruct((NP, D), jnp.float32),
    ],
    mesh=_mesh,
    compiler_params=pltpu.CompilerParams(use_tc_tiling_on_sc=False,
                                         needs_layout_passes=False),
    scratch_types=[
        pltpu.VMEM((N,), jnp.float32),
        pltpu.VMEM((N,), jnp.float32),
        pltpu.VMEM((B,), jnp.int32),
        pltpu.VMEM((B,), jnp.int32),
        pltpu.VMEM((B,), jnp.int32),
        pltpu.VMEM((B,), jnp.int32),
        pltpu.VMEM((B, D), jnp.float32),
        pltpu.VMEM((B, D), jnp.float32),
        pltpu.VMEM((B,), jnp.float32),
        pltpu.VMEM((B, 16), jnp.float32),
        pltpu.SemaphoreType.DMA,
        pltpu.SemaphoreType.DMA,
        pltpu.VMEM_SHARED((NP, D), jnp.float32),
        pltpu.VMEM_SHARED((NP, 16), jnp.float32),
    ],
)
def _edge(h1, l1, e1, s1, d1, f1, h2, l2, e2, s2, d2, f2, o1, o2,
          el_v, er_v, sidx0, sidx1, didx0, didx1, rows0, rows1, ex_v, exw_v,
          gsem0, gsem1, u_sh, s_sh):
    c = lax.axis_index("c")
    s = lax.axis_index("s")
    z16 = jnp.zeros((16,), jnp.float32)
    bufs = ((sidx0, didx0, rows0, gsem0), (sidx1, didx1, rows1, gsem1))

    def do_conv(a_h, el_h, er_h, src_h, dst_h, feat_h, out_h):
        pltpu.sync_copy(el_h, el_v)
        pltpu.sync_copy(er_h, er_v)

        @plsc.parallel_loop(0, B, unroll=4)
        def _(r):
            for j in range(D // 16):
                rows0[r, pl.ds(j * 16, 16)] = z16
            exw_v[r, pl.ds(0, 16)] = z16
        base = s * OPT
        for k in range(OPT // B):
            pltpu.sync_copy(rows0, u_sh.at[pl.ds(base + k * B, B)])
            pltpu.sync_copy(exw_v, s_sh.at[pl.ds(base + k * B, B)])

        def start_batch(i, p):
            si, di, rv, gs = bufs[p]
            row = s * RPT + i

            @pl.when((i < RPT) & (row < VROWS))
            def _():
                pltpu.sync_copy(src_h.at[row], si)
                pltpu.sync_copy(dst_h.at[row], di)
                pltpu.async_copy(a_h.at[si], rv, gs)

        def finish_batch(i, p):
            si, di, rv, gs = bufs[p]
            row = s * RPT + i

            @pl.when((i < RPT) & (row < VROWS))
            def _():
                pltpu.make_async_copy(a_h.at[pl.ds(0, B)], rv, gs).wait()
                for j in range(B // 16):
                    sl = pl.ds(j * 16, 16)
                    el16 = plsc.load_gather(el_v, [si[sl]])
                    er16 = plsc.load_gather(er_v, [di[sl]])
                    e = el16 + er16
                    e = jnp.where(e >= 0.0, e, 0.2 * e)
                    ex_v[sl] = jnp.exp(e)

                @plsc.parallel_loop(0, B)
                def _(r):
                    exv = plsc.load_gather(ex_v, [jnp.full((16,), r, jnp.int32)])
                    for j in range(D // 16):
                        sl = pl.ds(j * 16, 16)
                        rv[r, sl] = rv[r, sl] * exv
                    exw_v[r, pl.ds(0, 16)] = exv
                pltpu.sync_copy(rv, u_sh.at[di], add=True)
                pltpu.sync_copy(exw_v, s_sh.at[di], add=True)

        start_batch(0, 0)
        plsc.subcore_barrier()

        def pair(g, carry):
            i0 = 2 * g
            for p in range(2):
                start_batch(i0 + p + 1, 1 - p)
                finish_batch(i0 + p, p)
            return carry
        lax.fori_loop(0, NPAIR, pair, 0)
        plsc.subcore_barrier()

        def echunk(k, carry):
            rb = base + k * ECH
            pltpu.sync_copy(u_sh.at[pl.ds(rb, ECH)], rows0)
            pltpu.sync_copy(s_sh.at[pl.ds(rb, ECH)], exw_v)
            pltpu.sync_copy(feat_h.at[pl.ds(rb, ECH)], rows1)

            @plsc.parallel_loop(0, ECH)
            def _(r):
                iv = 1.0 / (exw_v[r, pl.ds(0, 16)] + 1e-9)
                for j in range(D // 16):
                    sl = pl.ds(j * 16, 16)
                    v = rows0[r, sl] * iv + rows1[r, sl]
                    rows0[r, sl] = jnp.where(v > 0.0, v, jnp.exp(v) - 1.0)
            pltpu.sync_copy(rows0, out_h.at[pl.ds(rb, ECH)])
            return carry
        lax.fori_loop(0, OPT // ECH, echunk, 0)

    @pl.when(c == 0)
    def _():
        do_conv(h1, l1, e1, s1, d1, f1, o1)

    @pl.when(c == 1)
    def _():
        do_conv(h2, l2, e2, s2, d2, f2, o2)


def _pad_edges(idx):
    idx = idx.astype(jnp.int32)
    padlen = EROWS * B - E
    pad = jnp.arange(padlen, dtype=jnp.int32) % N
    return jnp.concatenate([idx, pad]).reshape(EROWS, B)


def kernel(x_image, x_text, edge_index_image, edge_index_text,
           W_img, a_l_img, a_r_img, b_img, W_txt, a_l_txt, a_r_txt, b_txt):
    h1, h2, el1, er1, el2, er2, f1, f2 = _prep(
        x_image, x_text,
        W_img, a_l_img.reshape(D, 1), a_r_img.reshape(D, 1),
        b_img.reshape(1, D),
        W_txt, a_l_txt.reshape(D, 1), a_r_txt.reshape(D, 1),
        b_txt.reshape(1, D),
    )
    s1 = _pad_edges(edge_index_image[0])
    d1 = _pad_edges(edge_index_image[1])
    s2 = _pad_edges(edge_index_text[0])
    d2 = _pad_edges(edge_index_text[1])
    fpad = jnp.zeros((NP - N, D), jnp.float32)
    ftp = jnp.concatenate([f1, fpad])
    fip = jnp.concatenate([f2, fpad])
    new_text, new_image = _edge(
        h1, el1.reshape(N), er1.reshape(N), s1, d1, ftp,
        h2, el2.reshape(N), er2.reshape(N), s2, d2, fip,
    )
    return (new_image[:N], new_text[:N])


# speedup vs baseline: 32.3220x; 1.3554x over previous
import functools

import jax
import jax.numpy as jnp
from jax import lax
from jax.experimental import pallas as pl
from jax.experimental.pallas import tpu as pltpu
from jax.experimental.pallas import tpu_sc as plsc

N = 10000
D = 128
E = 160000
B = 64
NT = 16
VROWS = E // B
RPT = 157
EROWS = RPT * NT
NP = 10240
OPT = NP // NT
ECH = 64


def _prep_body(xi, xt, wi, ali, ari, bi, wt, alt, art, bt,
               h1, h2, el1, er1, el2, er2, f1, f2):
    ones16 = jnp.ones((1, 16), jnp.float32)
    h1[...] = jnp.dot(xi[...], wi[...], preferred_element_type=jnp.float32)
    el1[...] = jnp.dot(h1[...], ali[...],
                       preferred_element_type=jnp.float32) * ones16
    er1[...] = jnp.dot(xt[...], jnp.dot(wi[...], ari[...]),
                       preferred_element_type=jnp.float32) * ones16
    f1[...] = xt[...] + bi[...]
    h2[...] = jnp.dot(xt[...], wt[...], preferred_element_type=jnp.float32)
    el2[...] = jnp.dot(h2[...], alt[...],
                       preferred_element_type=jnp.float32) * ones16
    er2[...] = jnp.dot(xi[...], jnp.dot(wt[...], art[...]),
                       preferred_element_type=jnp.float32) * ones16
    f2[...] = xi[...] + bt[...]


_prep = pl.pallas_call(
    _prep_body,
    out_shape=[
        jax.ShapeDtypeStruct((N, D), jnp.float32),
        jax.ShapeDtypeStruct((N, D), jnp.float32),
        jax.ShapeDtypeStruct((N, 16), jnp.float32),
        jax.ShapeDtypeStruct((N, 16), jnp.float32),
        jax.ShapeDtypeStruct((N, 16), jnp.float32),
        jax.ShapeDtypeStruct((N, 16), jnp.float32),
        jax.ShapeDtypeStruct((N, D), jnp.float32),
        jax.ShapeDtypeStruct((N, D), jnp.float32),
    ],
)

_mesh = plsc.VectorSubcoreMesh(core_axis_name="c", subcore_axis_name="s")


@functools.partial(
    pl.kernel,
    out_type=[
        jax.ShapeDtypeStruct((NP, D), jnp.float32),
        jax.ShapeDtypeStruct((NP, D), jnp.float32),
    ],
    mesh=_mesh,
    compiler_params=pltpu.CompilerParams(use_tc_tiling_on_sc=False,
                                         needs_layout_passes=False),
    scratch_types=[
        pltpu.VMEM((3, 2, B), jnp.int32),
        pltpu.VMEM((B, D), jnp.float32),
        pltpu.VMEM((B, D), jnp.float32),
        pltpu.VMEM((B, D), jnp.float32),
        pltpu.VMEM((3, B, 16), jnp.float32),
        pltpu.VMEM((3, B, 16), jnp.float32),
        pltpu.VMEM((3, B, 16), jnp.float32),
        pltpu.SemaphoreType.DMA,
        pltpu.SemaphoreType.DMA,
        pltpu.SemaphoreType.DMA,
        pltpu.SemaphoreType.DMA,
        pltpu.SemaphoreType.DMA,
        pltpu.SemaphoreType.DMA,
        pltpu.VMEM_SHARED((NP, D), jnp.float32),
        pltpu.VMEM_SHARED((NP, 16), jnp.float32),
    ],
)
def _edge(h1, l1, e1, sd1, f1, h2, l2, e2, sd2, f2, o1, o2,
          idx_v, rows0, rows1, rows2, elg_v, erg_v, exw_v,
          gsem0, gsem1, gsem2, ssem0, ssem1, ssem2, u_sh, s_sh):
    c = lax.axis_index("c")
    s = lax.axis_index("s")
    z16 = jnp.zeros((16,), jnp.float32)
    rows = (rows0, rows1, rows2)
    gsems = (gsem0, gsem1, gsem2)
    ssems = (ssem0, ssem1, ssem2)

    def do_conv(a_h, el_h, er_h, sd_h, feat_h, out_h):
        base = s * OPT

        @plsc.parallel_loop(0, B, unroll=4)
        def _(r):
            for j in range(D // 16):
                rows0[r, pl.ds(j * 16, 16)] = z16
            exw_v[0, r, pl.ds(0, 16)] = z16
        for k in range(OPT // B):
            pltpu.sync_copy(rows0, u_sh.at[pl.ds(base + k * B, B)])
            pltpu.sync_copy(exw_v.at[0], s_sh.at[pl.ds(base + k * B, B)])

        def valid(i):
            return (i < RPT) & (s * RPT + i < VROWS)

        def load_idx_and_gather(i, p):
            @pl.when(valid(i))
            def _():
                pltpu.sync_copy(sd_h.at[s * RPT + i], idx_v.at[p])
                pltpu.async_copy(a_h.at[idx_v.at[p, 0]], rows[p], gsems[p])
                pltpu.async_copy(el_h.at[idx_v.at[p, 0]], elg_v.at[p],
                                 gsems[p])
                pltpu.async_copy(er_h.at[idx_v.at[p, 1]], erg_v.at[p],
                                 gsems[p])

        def step(i, p):
            rv = rows[p]

            @pl.when(valid(i))
            def _():
                pltpu.make_async_copy(a_h.at[pl.ds(0, B)], rv, gsems[p]).wait()
                pltpu.make_async_copy(el_h.at[pl.ds(0, B)], elg_v.at[p],
                                      gsems[p]).wait()
                pltpu.make_async_copy(er_h.at[pl.ds(0, B)], erg_v.at[p],
                                      gsems[p]).wait()

                @plsc.parallel_loop(0, B)
                def _(r):
                    e = elg_v[p, r, pl.ds(0, 16)] + erg_v[p, r, pl.ds(0, 16)]
                    e = jnp.where(e >= 0.0, e, 0.2 * e)
                    exv = jnp.exp(e)
                    for j in range(D // 16):
                        sl = pl.ds(j * 16, 16)
                        rv[r, sl] = rv[r, sl] * exv
                    exw_v[p, r, pl.ds(0, 16)] = exv
                pltpu.async_copy(rv, u_sh.at[idx_v.at[p, 1]], ssems[p],
                                 add=True)
                pltpu.async_copy(exw_v.at[p], s_sh.at[idx_v.at[p, 1]],
                                 ssems[p], add=True)

            @pl.when((i >= 1) & valid(i - 1))
            def _():
                q = (p + 2) % 3
                pltpu.make_async_copy(a_h.at[pl.ds(0, B)], rows[q],
                                      ssems[q]).wait()
                pltpu.make_async_copy(el_h.at[pl.ds(0, B)], exw_v.at[q],
                                      ssems[q]).wait()
            load_idx_and_gather(i + 2, (p + 2) % 3)

        load_idx_and_gather(0, 0)
        load_idx_and_gather(1, 1)
        plsc.subcore_barrier()

        def trip(g, carry):
            i0 = 3 * g
            for p in range(3):
                step(i0 + p, p)
            return carry
        lax.fori_loop(0, (RPT + 2) // 3, trip, 0)
        plsc.subcore_barrier()

        def echunk(k, carry):
            rb = base + k * ECH
            pltpu.sync_copy(u_sh.at[pl.ds(rb, ECH)], rows0)
            pltpu.sync_copy(s_sh.at[pl.ds(rb, ECH)], exw_v.at[0])
            pltpu.sync_copy(feat_h.at[pl.ds(rb, ECH)], rows1)

            @plsc.parallel_loop(0, ECH)
            def _(r):
                iv = 1.0 / (exw_v[0, r, pl.ds(0, 16)] + 1e-9)
                for j in range(D // 16):
                    sl = pl.ds(j * 16, 16)
                    v = rows0[r, sl] * iv + rows1[r, sl]
                    rows0[r, sl] = jnp.where(v > 0.0, v, jnp.exp(v) - 1.0)
            pltpu.sync_copy(rows0, out_h.at[pl.ds(rb, ECH)])
            return carry
        lax.fori_loop(0, OPT // ECH, echunk, 0)

    @pl.when(c == 0)
    def _():
        do_conv(h1, l1, e1, sd1, f1, o1)

    @pl.when(c == 1)
    def _():
        do_conv(h2, l2, e2, sd2, f2, o2)


def _pad_edges(ei):
    idx = ei.astype(jnp.int32)
    padlen = EROWS * B - E
    pad = jnp.arange(padlen, dtype=jnp.int32) % N
    src = jnp.concatenate([idx[0], pad]).reshape(EROWS, 1, B)
    dst = jnp.concatenate([idx[1], pad]).reshape(EROWS, 1, B)
    return jnp.concatenate([src, dst], axis=1)


def kernel(x_image, x_text, edge_index_image, edge_index_text,
           W_img, a_l_img, a_r_img, b_img, W_txt, a_l_txt, a_r_txt, b_txt):
    h1, h2, el1, er1, el2, er2, f1, f2 = _prep(
        x_image, x_text,
        W_img, a_l_img.reshape(D, 1), a_r_img.reshape(D, 1),
        b_img.reshape(1, D),
        W_txt, a_l_txt.reshape(D, 1), a_r_txt.reshape(D, 1),
        b_txt.reshape(1, D),
    )
    sd1 = _pad_edges(edge_index_image)
    sd2 = _pad_edges(edge_index_text)
    fpad = jnp.zeros((NP - N, D), jnp.float32)
    ftp = jnp.concatenate([f1, fpad])
    fip = jnp.concatenate([f2, fpad])
    new_text, new_image = _edge(
        h1, el1, er1, sd1, ftp,
        h2, el2, er2, sd2, fip,
    )
    return (new_image[:N], new_text[:N])


# speedup vs baseline: 35.9898x; 1.1135x over previous
import functools

import jax
import jax.numpy as jnp
from jax import lax
from jax.experimental import pallas as pl
from jax.experimental.pallas import tpu as pltpu
from jax.experimental.pallas import tpu_sc as plsc

N = 10000
D = 128
E = 160000
B = 64
NT = 16
VROWS = E // B
RPT = 157
NP = 10240
OPT = NP // NT
ECH = 64
TAILRB = (N // ECH) * ECH
NTAIL = N - TAILRB
TAILS = N // OPT


def _prep_body(xi, xt, wi, ali, ari, wt, alt, art,
               h1, h2, el1, er1, el2, er2):
    ones16 = jnp.ones((1, 16), jnp.float32)
    h1[...] = jnp.dot(xi[...], wi[...], preferred_element_type=jnp.float32)
    el1[...] = jnp.dot(h1[...], ali[...],
                       preferred_element_type=jnp.float32) * ones16
    er1[...] = jnp.dot(xt[...], jnp.dot(wi[...], ari[...]),
                       preferred_element_type=jnp.float32) * ones16
    h2[...] = jnp.dot(xt[...], wt[...], preferred_element_type=jnp.float32)
    el2[...] = jnp.dot(h2[...], alt[...],
                       preferred_element_type=jnp.float32) * ones16
    er2[...] = jnp.dot(xi[...], jnp.dot(wt[...], art[...]),
                       preferred_element_type=jnp.float32) * ones16


_prep = pl.pallas_call(
    _prep_body,
    out_shape=[
        jax.ShapeDtypeStruct((N, D), jnp.float32),
        jax.ShapeDtypeStruct((N, D), jnp.float32),
        jax.ShapeDtypeStruct((N, 16), jnp.float32),
        jax.ShapeDtypeStruct((N, 16), jnp.float32),
        jax.ShapeDtypeStruct((N, 16), jnp.float32),
        jax.ShapeDtypeStruct((N, 16), jnp.float32),
    ],
)

_mesh = plsc.VectorSubcoreMesh(core_axis_name="c", subcore_axis_name="s")


@functools.partial(
    pl.kernel,
    out_type=[
        jax.ShapeDtypeStruct((N, D), jnp.float32),
        jax.ShapeDtypeStruct((N, D), jnp.float32),
    ],
    mesh=_mesh,
    compiler_params=pltpu.CompilerParams(use_tc_tiling_on_sc=False,
                                         needs_layout_passes=False),
    scratch_types=[
        pltpu.VMEM((3, 2, B), jnp.int32),
        pltpu.VMEM((B, D), jnp.float32),
        pltpu.VMEM((B, D), jnp.float32),
        pltpu.VMEM((B, D), jnp.float32),
        pltpu.VMEM((3, B, 16), jnp.float32),
        pltpu.VMEM((3, B, 16), jnp.float32),
        pltpu.VMEM((3, B, 16), jnp.float32),
        pltpu.VMEM((1, D), jnp.float32),
        pltpu.SemaphoreType.DMA,
        pltpu.SemaphoreType.DMA,
        pltpu.SemaphoreType.DMA,
        pltpu.SemaphoreType.DMA,
        pltpu.SemaphoreType.DMA,
        pltpu.SemaphoreType.DMA,
        pltpu.VMEM_SHARED((NP, D), jnp.float32),
        pltpu.VMEM_SHARED((NP, 16), jnp.float32),
    ],
)
def _edge(h1, l1, e1, sd1, f1, b1, h2, l2, e2, sd2, f2, b2, o1, o2,
          idx_v, rows0, rows1, rows2, elg_v, erg_v, exw_v, bias_v,
          gsem0, gsem1, gsem2, ssem0, ssem1, ssem2, u_sh, s_sh):
    c = lax.axis_index("c")
    s = lax.axis_index("s")
    z16 = jnp.zeros((16,), jnp.float32)
    rows = (rows0, rows1, rows2)
    gsems = (gsem0, gsem1, gsem2)
    ssems = (ssem0, ssem1, ssem2)

    def do_conv(a_h, el_h, er_h, sd_h, feat_h, b_h, out_h):
        base = s * OPT
        pltpu.sync_copy(b_h, bias_v)

        @plsc.parallel_loop(0, B, unroll=4)
        def _(r):
            for j in range(D // 16):
                rows0[r, pl.ds(j * 16, 16)] = z16
            exw_v[0, r, pl.ds(0, 16)] = z16
        for k in range(OPT // B):
            pltpu.sync_copy(rows0, u_sh.at[pl.ds(base + k * B, B)])
            pltpu.sync_copy(exw_v.at[0], s_sh.at[pl.ds(base + k * B, B)])

        def valid(i):
            return (i < RPT) & (s * RPT + i < VROWS)

        def load_idx_and_gather(i, p):
            @pl.when(valid(i))
            def _():
                pltpu.sync_copy(sd_h.at[:, s * RPT + i], idx_v.at[p])
                pltpu.async_copy(a_h.at[idx_v.at[p, 0]], rows[p], gsems[p])
                pltpu.async_copy(el_h.at[idx_v.at[p, 0]], elg_v.at[p],
                                 gsems[p])
                pltpu.async_copy(er_h.at[idx_v.at[p, 1]], erg_v.at[p],
                                 gsems[p])

        def step(i, p):
            rv = rows[p]

            @pl.when(valid(i))
            def _():
                pltpu.make_async_copy(a_h.at[pl.ds(0, B)], rv, gsems[p]).wait()
                pltpu.make_async_copy(el_h.at[pl.ds(0, B)], elg_v.at[p],
                                      gsems[p]).wait()
                pltpu.make_async_copy(er_h.at[pl.ds(0, B)], erg_v.at[p],
                                      gsems[p]).wait()

                @plsc.parallel_loop(0, B)
                def _(r):
                    e = elg_v[p, r, pl.ds(0, 16)] + erg_v[p, r, pl.ds(0, 16)]
                    e = jnp.where(e >= 0.0, e, 0.2 * e)
                    exv = jnp.exp(e)
                    for j in range(D // 16):
                        sl = pl.ds(j * 16, 16)
                        rv[r, sl] = rv[r, sl] * exv
                    exw_v[p, r, pl.ds(0, 16)] = exv
                pltpu.async_copy(rv, u_sh.at[idx_v.at[p, 1]], ssems[p],
                                 add=True)
                pltpu.async_copy(exw_v.at[p], s_sh.at[idx_v.at[p, 1]],
                                 ssems[p], add=True)

            @pl.when((i >= 1) & valid(i - 1))
            def _():
                q = (p + 2) % 3
                pltpu.make_async_copy(a_h.at[pl.ds(0, B)], rows[q],
                                      ssems[q]).wait()
                pltpu.make_async_copy(el_h.at[pl.ds(0, B)], exw_v.at[q],
                                      ssems[q]).wait()
            load_idx_and_gather(i + 2, (p + 2) % 3)

        load_idx_and_gather(0, 0)
        load_idx_and_gather(1, 1)
        plsc.subcore_barrier()

        def trip(g, carry):
            i0 = 3 * g
            for p in range(3):
                step(i0 + p, p)
            return carry
        lax.fori_loop(0, (RPT + 2) // 3, trip, 0)
        plsc.subcore_barrier()

        def echunk(rb, nr):
            pltpu.sync_copy(u_sh.at[pl.ds(rb, nr)], rows0.at[pl.ds(0, nr)])
            pltpu.sync_copy(s_sh.at[pl.ds(rb, nr)],
                            exw_v.at[0, pl.ds(0, nr)])
            pltpu.sync_copy(feat_h.at[pl.ds(rb, nr)], rows1.at[pl.ds(0, nr)])

            @plsc.parallel_loop(0, nr)
            def _(r):
                iv = 1.0 / (exw_v[0, r, pl.ds(0, 16)] + 1e-9)
                for j in range(D // 16):
                    sl = pl.ds(j * 16, 16)
                    v = rows0[r, sl] * iv + rows1[r, sl] + bias_v[0, sl]
                    rows0[r, sl] = jnp.where(v > 0.0, v, jnp.exp(v) - 1.0)
            pltpu.sync_copy(rows0.at[pl.ds(0, nr)], out_h.at[pl.ds(rb, nr)])

        def full_chunk(k, carry):
            rb = base + k * ECH

            @pl.when(rb + ECH <= N)
            def _():
                echunk(rb, ECH)
            return carry
        lax.fori_loop(0, OPT // ECH, full_chunk, 0)

        @pl.when(s == TAILS)
        def _():
            echunk(TAILRB, NTAIL)

    @pl.when(c == 0)
    def _():
        do_conv(h1, l1, e1, sd1, f1, b1, o1)

    @pl.when(c == 1)
    def _():
        do_conv(h2, l2, e2, sd2, f2, b2, o2)


def kernel(x_image, x_text, edge_index_image, edge_index_text,
           W_img, a_l_img, a_r_img, b_img, W_txt, a_l_txt, a_r_txt, b_txt):
    h1, h2, el1, er1, el2, er2 = _prep(
        x_image, x_text,
        W_img, a_l_img.reshape(D, 1), a_r_img.reshape(D, 1),
        W_txt, a_l_txt.reshape(D, 1), a_r_txt.reshape(D, 1),
    )
    sd1 = edge_index_image.astype(jnp.int32).reshape(2, VROWS, B)
    sd2 = edge_index_text.astype(jnp.int32).reshape(2, VROWS, B)
    new_text, new_image = _edge(
        h1, el1, er1, sd1, x_text, b_img.reshape(1, D),
        h2, el2, er2, sd2, x_image, b_txt.reshape(1, D),
    )
    return (new_image, new_text)


# speedup vs baseline: 39.3657x; 1.0938x over previous
import functools

import jax
import jax.numpy as jnp
from jax import lax
from jax.experimental import pallas as pl
from jax.experimental.pallas import tpu as pltpu
from jax.experimental.pallas import tpu_sc as plsc

N = 10000
D = 128
E = 160000
B = 64
NT = 16
VROWS = E // B
GSZ = 6
RPT = 162
GN = RPT // GSZ
SROWS = RPT * NT
NP = 10240
OPT = NP // NT
ECH = 64
TAILRB = (N // ECH) * ECH
NTAIL = N - TAILRB
TAILS = N // OPT


def _prep_body(xi, xt, wi, ali, ari, wt, alt, art,
               h1, h2, el1, er1, el2, er2):
    ones16 = jnp.ones((1, 16), jnp.float32)
    h1[...] = jnp.dot(xi[...], wi[...], preferred_element_type=jnp.float32)
    el1[...] = jnp.dot(h1[...], ali[...],
                       preferred_element_type=jnp.float32) * ones16
    er1[...] = jnp.dot(xt[...], jnp.dot(wi[...], ari[...]),
                       preferred_element_type=jnp.float32) * ones16
    h2[...] = jnp.dot(xt[...], wt[...], preferred_element_type=jnp.float32)
    el2[...] = jnp.dot(h2[...], alt[...],
                       preferred_element_type=jnp.float32) * ones16
    er2[...] = jnp.dot(xi[...], jnp.dot(wt[...], art[...]),
                       preferred_element_type=jnp.float32) * ones16


_prep = pl.pallas_call(
    _prep_body,
    out_shape=[
        jax.ShapeDtypeStruct((N, D), jnp.float32),
        jax.ShapeDtypeStruct((N, D), jnp.float32),
        jax.ShapeDtypeStruct((N, 16), jnp.float32),
        jax.ShapeDtypeStruct((N, 16), jnp.float32),
        jax.ShapeDtypeStruct((N, 16), jnp.float32),
        jax.ShapeDtypeStruct((N, 16), jnp.float32),
    ],
)

_mesh = plsc.VectorSubcoreMesh(core_axis_name="c", subcore_axis_name="s")


@functools.partial(
    pl.kernel,
    out_type=[
        jax.ShapeDtypeStruct((N, D), jnp.float32),
        jax.ShapeDtypeStruct((N, D), jnp.float32),
    ],
    mesh=_mesh,
    compiler_params=pltpu.CompilerParams(use_tc_tiling_on_sc=False,
                                         needs_layout_passes=False),
    scratch_types=[
        pltpu.VMEM((2, 2, GSZ, B), jnp.int32),
        pltpu.VMEM((B, D), jnp.float32),
        pltpu.VMEM((B, D), jnp.float32),
        pltpu.VMEM((B, D), jnp.float32),
        pltpu.VMEM((3, B, 16), jnp.float32),
        pltpu.VMEM((3, B, 16), jnp.float32),
        pltpu.VMEM((3, B, 16), jnp.float32),
        pltpu.VMEM((1, D), jnp.float32),
        pltpu.SemaphoreType.DMA,
        pltpu.SemaphoreType.DMA,
        pltpu.SemaphoreType.DMA,
        pltpu.SemaphoreType.DMA,
        pltpu.SemaphoreType.DMA,
        pltpu.SemaphoreType.DMA,
        pltpu.VMEM_SHARED((NP, D), jnp.float32),
        pltpu.VMEM_SHARED((NP, 16), jnp.float32),
    ],
)
def _edge(h1, l1, e1, sd1, f1, b1, h2, l2, e2, sd2, f2, b2, o1, o2,
          idx_v, rows0, rows1, rows2, elg_v, erg_v, exw_v, bias_v,
          gsem0, gsem1, gsem2, ssem0, ssem1, ssem2, u_sh, s_sh):
    c = lax.axis_index("c")
    s = lax.axis_index("s")
    z16 = jnp.zeros((16,), jnp.float32)
    rows = (rows0, rows1, rows2)
    gsems = (gsem0, gsem1, gsem2)
    ssems = (ssem0, ssem1, ssem2)

    def do_conv(a_h, el_h, er_h, sd_h, feat_h, b_h, out_h):
        base = s * OPT
        pltpu.sync_copy(b_h, bias_v)

        @plsc.parallel_loop(0, B, unroll=4)
        def _(r):
            for j in range(D // 16):
                rows0[r, pl.ds(j * 16, 16)] = z16
            exw_v[0, r, pl.ds(0, 16)] = z16
        for k in range(OPT // B):
            pltpu.sync_copy(rows0, u_sh.at[pl.ds(base + k * B, B)])
            pltpu.sync_copy(exw_v.at[0], s_sh.at[pl.ds(base + k * B, B)])

        def valid(i):
            return (i < RPT) & (s * RPT + i < VROWS)

        def load_group(g, gb):
            g0 = jnp.minimum(s * RPT + g * GSZ, SROWS - GSZ)
            pltpu.sync_copy(sd_h.at[:, pl.ds(g0, GSZ)], idx_v.at[gb])

        def issue_gather(i, p, gbuf, off):
            @pl.when(valid(i))
            def _():
                pltpu.async_copy(a_h.at[idx_v.at[gbuf, 0, off]], rows[p],
                                 gsems[p])
                pltpu.async_copy(el_h.at[idx_v.at[gbuf, 0, off]],
                                 elg_v.at[p], gsems[p])
                pltpu.async_copy(er_h.at[idx_v.at[gbuf, 1, off]],
                                 erg_v.at[p], gsems[p])

        def step(i, k, gb, gb2):
            p = k % 3
            rv = rows[p]

            @pl.when(valid(i))
            def _():
                pltpu.make_async_copy(a_h.at[pl.ds(0, B)], rv, gsems[p]).wait()
                pltpu.make_async_copy(el_h.at[pl.ds(0, B)], elg_v.at[p],
                                      gsems[p]).wait()
                pltpu.make_async_copy(er_h.at[pl.ds(0, B)], erg_v.at[p],
                                      gsems[p]).wait()

                @plsc.parallel_loop(0, B)
                def _(r):
                    e = elg_v[p, r, pl.ds(0, 16)] + erg_v[p, r, pl.ds(0, 16)]
                    e = jnp.where(e >= 0.0, e, 0.2 * e)
                    exv = jnp.exp(e)
                    for j in range(D // 16):
                        sl = pl.ds(j * 16, 16)
                        rv[r, sl] = rv[r, sl] * exv
                    exw_v[p, r, pl.ds(0, 16)] = exv
                pltpu.async_copy(rv, u_sh.at[idx_v.at[gb, 1, k]], ssems[p],
                                 add=True)
                pltpu.async_copy(exw_v.at[p], s_sh.at[idx_v.at[gb, 1, k]],
                                 ssems[p], add=True)

            @pl.when((i >= 1) & valid(i - 1))
            def _():
                q = (p + 2) % 3
                pltpu.make_async_copy(a_h.at[pl.ds(0, B)], rows[q],
                                      ssems[q]).wait()
                pltpu.make_async_copy(el_h.at[pl.ds(0, B)], exw_v.at[q],
                                      ssems[q]).wait()
            if k < 4:
                issue_gather(i + 2, (p + 2) % 3, gb, k + 2)
            else:
                issue_gather(i + 2, (p + 2) % 3, gb2, k - 4)

        load_group(0, 0)
        issue_gather(0, 0, 0, 0)
        issue_gather(1, 1, 0, 1)
        plsc.subcore_barrier()

        def group(g, carry):
            gb = lax.rem(g, 2)
            gb2 = 1 - gb
            i0 = g * GSZ
            step(i0, 0, gb, gb2)
            load_group(g + 1, gb2)
            for k in range(1, GSZ):
                step(i0 + k, k, gb, gb2)
            return carry
        lax.fori_loop(0, GN, group, 0)
        @pl.when(valid(RPT - 1))
        def _():
            q = (RPT - 1) % 3
            pltpu.make_async_copy(a_h.at[pl.ds(0, B)], rows[q],
                                  ssems[q]).wait()
            pltpu.make_async_copy(el_h.at[pl.ds(0, B)], exw_v.at[q],
                                  ssems[q]).wait()
        plsc.subcore_barrier()

        def echunk(rb, nr):
            pltpu.sync_copy(u_sh.at[pl.ds(rb, nr)], rows0.at[pl.ds(0, nr)])
            pltpu.sync_copy(s_sh.at[pl.ds(rb, nr)],
                            exw_v.at[0, pl.ds(0, nr)])
            pltpu.sync_copy(feat_h.at[pl.ds(rb, nr)], rows1.at[pl.ds(0, nr)])

            @plsc.parallel_loop(0, nr)
            def _(r):
                iv = 1.0 / (exw_v[0, r, pl.ds(0, 16)] + 1e-9)
                for j in range(D // 16):
                    sl = pl.ds(j * 16, 16)
                    v = rows0[r, sl] * iv + rows1[r, sl] + bias_v[0, sl]
                    rows0[r, sl] = jnp.where(v > 0.0, v, jnp.exp(v) - 1.0)
            pltpu.sync_copy(rows0.at[pl.ds(0, nr)], out_h.at[pl.ds(rb, nr)])

        def full_chunk(k, carry):
            rb = base + k * ECH

            @pl.when(rb + ECH <= N)
            def _():
                echunk(rb, ECH)
            return carry
        lax.fori_loop(0, OPT // ECH, full_chunk, 0)

        @pl.when(s == TAILS)
        def _():
            echunk(TAILRB, NTAIL)

    @pl.when(c == 0)
    def _():
        do_conv(h1, l1, e1, sd1, f1, b1, o1)

    @pl.when(c == 1)
    def _():
        do_conv(h2, l2, e2, sd2, f2, b2, o2)


def kernel(x_image, x_text, edge_index_image, edge_index_text,
           W_img, a_l_img, a_r_img, b_img, W_txt, a_l_txt, a_r_txt, b_txt):
    h1, h2, el1, er1, el2, er2 = _prep(
        x_image, x_text,
        W_img, a_l_img.reshape(D, 1), a_r_img.reshape(D, 1),
        W_txt, a_l_txt.reshape(D, 1), a_r_txt.reshape(D, 1),
    )
    pad = ((0, 0), (0, SROWS - VROWS), (0, 0))
    sd1 = jnp.pad(edge_index_image.astype(jnp.int32).reshape(2, VROWS, B),
                  pad)
    sd2 = jnp.pad(edge_index_text.astype(jnp.int32).reshape(2, VROWS, B),
                  pad)
    new_text, new_image = _edge(
        h1, el1, er1, sd1, x_text, b_img.reshape(1, D),
        h2, el2, er2, sd2, x_image, b_txt.reshape(1, D),
    )
    return (new_image, new_text)
